# baseline (speedup 1.0000x reference)
import jax
import jax.numpy as jnp
from jax.experimental import pallas as pl

N = 1_000_000
N_THR = 19
N_COLS = 21
BLOCK = 8000


def _body(t_ref, f_ref, o_ref):
    f = f_ref[:]
    t = t_ref[:]
    gt = (f > t).astype(jnp.int32)
    bins = jnp.sum(gt, axis=1, keepdims=True)
    cols = jax.lax.broadcasted_iota(jnp.int32, (BLOCK, N_COLS), 1)
    o_ref[:, :] = (cols == bins).astype(jnp.int32)


def kernel(feature, thresholds):
    f2 = feature.reshape(N, 1)
    t2 = thresholds.reshape(1, N_THR)
    grid = N // BLOCK
    return pl.pallas_call(
        _body,
        grid=(grid,),
        in_specs=[
            pl.BlockSpec((1, N_THR), lambda i: (0, 0)),
            pl.BlockSpec((BLOCK, 1), lambda i: (i, 0)),
        ],
        out_specs=pl.BlockSpec((BLOCK, N_COLS), lambda i: (i, 0)),
        out_shape=jax.ShapeDtypeStruct((N, N_COLS), jnp.int32),
    )(t2, f2)


# speedup vs baseline: 1.3239x; 1.3239x over previous
import numpy as np
import jax
import jax.numpy as jnp
from jax.experimental import pallas as pl
from jax.experimental.pallas import tpu as pltpu

N = 1_000_000
N_THR = 19
N_COLS = 21
R = 125
G = N // R
L = R * N_COLS
BG = 400
GRID = G // BG

_REP = np.zeros((R, L), dtype=np.float32)
for _r in range(R):
    _REP[_r, _r * N_COLS:(_r + 1) * N_COLS] = 1.0
_CMOD = (np.arange(L, dtype=np.int64) % N_COLS).astype(np.float32)[None, :]


def _body(t_ref, f_ref, rep_ref, cmod_ref, o_ref):
    fv = f_ref[:]
    bins = jnp.zeros((BG, R), jnp.float32)
    for j in range(N_THR):
        tj = t_ref[0, j]
        bins = bins + jnp.where(fv > tj, 1.0, 0.0)
    binrep = jax.lax.dot_general(
        bins.astype(jnp.bfloat16), rep_ref[:],
        dimension_numbers=(((1,), (0,)), ((), ())),
        preferred_element_type=jnp.float32,
    )
    o_ref[:, :] = (binrep == cmod_ref[:]).astype(jnp.int32)


def kernel(feature, thresholds):
    fv = feature.reshape(G, R)
    t2 = thresholds.reshape(1, N_THR)
    rep = jnp.asarray(_REP, dtype=jnp.bfloat16)
    cmod = jnp.asarray(_CMOD)
    out2 = pl.pallas_call(
        _body,
        grid=(GRID,),
        in_specs=[
            pl.BlockSpec(memory_space=pltpu.SMEM),
            pl.BlockSpec((BG, R), lambda i: (i, 0)),
            pl.BlockSpec((R, L), lambda i: (0, 0)),
            pl.BlockSpec((1, L), lambda i: (0, 0)),
        ],
        out_specs=pl.BlockSpec((BG, L), lambda i: (i, 0)),
        out_shape=jax.ShapeDtypeStruct((G, L), jnp.int32),
    )(t2, fv, rep, cmod)
    return out2.reshape(N, N_COLS)


# speedup vs baseline: 2.2777x; 1.7205x over previous
import functools
import jax
import jax.numpy as jnp
from jax import lax
from jax.experimental import pallas as pl
from jax.experimental.pallas import tpu as pltpu
from jax.experimental.pallas import tpu_sc as plsc

N = 1_000_000
N_THR = 19
N_COLS = 21
CH = 400
NCHUNK = N // CH
NG = CH // 16
NW = 32
NPAIR = (NCHUNK + 2 * NW - 1) // (2 * NW)

_mesh = plsc.VectorSubcoreMesh(core_axis_name="c", subcore_axis_name="s")

_GDN = lax.GatherDimensionNumbers(
    offset_dims=(), collapsed_slice_dims=(0,), start_index_map=(0,)
)


def _take16(vec, j):
    idx = jnp.full((16, 1), j, jnp.int32) if isinstance(j, int) else j
    return lax.gather(vec, idx, _GDN, slice_sizes=(1,),
                      mode=lax.GatherScatterMode.PROMISE_IN_BOUNDS)


@functools.partial(
    pl.kernel,
    mesh=_mesh,
    out_type=jax.ShapeDtypeStruct((N, N_COLS), jnp.int32),
    scratch_types=[
        pltpu.VMEM((CH, N_COLS), jnp.int32),
        pltpu.VMEM((CH, N_COLS), jnp.int32),
        pltpu.VMEM((CH,), jnp.float32),
        pltpu.VMEM((CH,), jnp.float32),
        pltpu.VMEM((32,), jnp.float32),
        pltpu.SemaphoreType.DMA,
        pltpu.SemaphoreType.DMA,
        pltpu.SemaphoreType.DMA,
        pltpu.SemaphoreType.DMA,
    ],
)
def _sc_kernel(f_hbm, t_hbm, out_hbm,
               stage_a, stage_b, fb0, fb1, tvm,
               sem_oa, sem_ob, sem_f0, sem_f1):
    wid = lax.axis_index("s") * 2 + lax.axis_index("c")
    lanes = lax.iota(jnp.int32, 16)
    lanes_p5 = lanes + 5
    one16 = jnp.ones((16,), jnp.int32)
    z16 = jnp.zeros((16,), jnp.int32)

    pltpu.sync_copy(t_hbm, tvm.at[pl.ds(0, N_THR)])
    tv0 = tvm[pl.ds(0, 16)]
    tv1 = tvm[pl.ds(16, 16)]
    tsplat = [
        _take16(tv0 if j < 16 else tv1, j % 16)
        for j in range(N_THR)
    ]

    def compute(stage, fb):
        def grp(g, c):
            f = fb[pl.ds(16 * g, 16)]
            acc = z16
            for j in range(N_THR):
                acc = acc + jnp.where(f > tsplat[j], one16, z16)
            for r in range(16):
                b = _take16(acc, r)
                row = 16 * g + r
                stage[row, pl.ds(0, 16)] = jnp.where(lanes == b, one16, z16)
                stage[row, pl.ds(5, 16)] = jnp.where(lanes_p5 == b, one16, z16)
            return c

        lax.fori_loop(0, NG, grp, 0)

    def fetch(k, fb, sem):
        pltpu.make_async_copy(f_hbm.at[pl.ds(k * CH, CH)], fb, sem).start()

    def fetch_wait(fb, sem):
        pltpu.make_async_copy(f_hbm.at[pl.ds(0, CH)], fb, sem).wait()

    def out_start(stage, k, sem):
        pltpu.make_async_copy(
            stage,
            out_hbm.at[pl.ds(k * CH, CH), :],
            sem,
        ).start()

    def out_wait(stage, sem):
        pltpu.make_async_copy(
            stage,
            out_hbm.at[pl.ds(0, CH), :],
            sem,
        ).wait()

    fetch(wid, fb0, sem_f0)

    def pair(p, c):
        ka = wid + 64 * p
        kb = ka + 32

        @pl.when(kb < NCHUNK)
        def _():
            fetch(kb, fb1, sem_f1)

        @pl.when(ka < NCHUNK)
        def _():
            fetch_wait(fb0, sem_f0)

        @pl.when((p > 0) & (ka < NCHUNK))
        def _():
            out_wait(stage_a, sem_oa)

        @pl.when(ka < NCHUNK)
        def _():
            compute(stage_a, fb0)
            out_start(stage_a, ka, sem_oa)

        @pl.when(kb + 32 < NCHUNK)
        def _():
            fetch(kb + 32, fb0, sem_f0)

        @pl.when(kb < NCHUNK)
        def _():
            fetch_wait(fb1, sem_f1)

        @pl.when((p > 0) & (kb < NCHUNK))
        def _():
            out_wait(stage_b, sem_ob)

        @pl.when(kb < NCHUNK)
        def _():
            compute(stage_b, fb1)
            out_start(stage_b, kb, sem_ob)

        return c

    lax.fori_loop(0, NPAIR, pair, 0)
    out_wait(stage_a, sem_oa)
    out_wait(stage_b, sem_ob)


def kernel(feature, thresholds):
    return _sc_kernel(feature, thresholds)
